# Initial kernel scaffold; baseline (speedup 1.0000x reference)
import jax
import jax.numpy as jnp
from jax.experimental import pallas as pl


def kernel(X, H, params, codebooks):
    raise NotImplementedError("write your pallas kernel here")


# speedup vs baseline: 1.8592x; 1.8592x over previous
import functools

import jax
import jax.numpy as jnp
import numpy as np
from jax import lax
from jax.experimental import pallas as pl
from jax.experimental.pallas import tpu as pltpu

N_NODES = 10000
N_INC = 320000
D = 128
K = 512
L = 2
TAU = 1.0
CC = 0.25
BN_SCALE = float(1.0 / np.sqrt(1.0 + 1e-5))

R_A = 2000
R_B = 1000


def _conv_in_body(x_ref, g_ref, b_ref, w_ref, o_ref):
    h = g_ref[...] * (x_ref[...] * BN_SCALE) + b_ref[...]
    o_ref[...] = jnp.dot(h, w_ref[...])


def _conv_in(X, g, b, W):
    return pl.pallas_call(
        _conv_in_body,
        grid=(N_NODES // R_A,),
        in_specs=[
            pl.BlockSpec((R_A, D), lambda i: (i, 0)),
            pl.BlockSpec((1, D), lambda i: (0, 0)),
            pl.BlockSpec((1, D), lambda i: (0, 0)),
            pl.BlockSpec((D, D), lambda i: (0, 0)),
        ],
        out_specs=pl.BlockSpec((R_A, D), lambda i: (i, 0)),
        out_shape=jax.ShapeDtypeStruct((N_NODES, D), jnp.float32),
    )(X, g.reshape(1, D), b.reshape(1, D), W)


def _vq_body(conv_ref, x_ref, gum_ref, cb_ref, cbias_ref, gg_ref, gb_ref,
             gw_ref, gbias_ref, xo_ref, loss_ref, perp_ref,
             ll_acc, cnt_acc):
    i = pl.program_id(0)
    nblk = pl.num_programs(0)

    @pl.when(i == 0)
    def _init():
        ll_acc[0, 0] = jnp.float32(0.0)
        cnt_acc[...] = jnp.zeros_like(cnt_acc)

    h = jnp.maximum(conv_ref[...] + cbias_ref[...], 0.0)
    gx = gg_ref[...] * (x_ref[...] * BN_SCALE) + gb_ref[...]
    glogit = jnp.sum(gx * gw_ref[...], axis=1, keepdims=True) + gbias_ref[...]
    gate = jax.nn.sigmoid(glogit)
    msg = h * gate

    cb = cb_ref[...]
    m2 = jnp.sum(msg * msg, axis=1, keepdims=True)
    c2 = jnp.sum(cb * cb, axis=1)
    gmat = lax.dot_general(msg, cb, (((1,), (1,)), ((), ())))
    dist = m2 + c2[None, :] - 2.0 * gmat

    s = -dist
    smax = jnp.max(s, axis=1, keepdims=True)
    e = jnp.exp(s - smax)
    z = jnp.sum(e, axis=1, keepdims=True)
    soft = e / z
    ll_rows = jnp.sum(soft * jnp.log(jnp.maximum(soft, 1e-8)), axis=1)
    ll_acc[0, 0] += jnp.sum(ll_rows)

    score = s + gum_ref[...]
    mx = jnp.max(score, axis=1, keepdims=True)
    iota = lax.broadcasted_iota(jnp.int32, score.shape, 1)
    cand = jnp.where(score == mx, iota, K)
    idx = jnp.min(cand, axis=1, keepdims=True)
    enc = (iota == idx).astype(jnp.float32)

    quant = jnp.dot(enc, cb)
    cnt_acc[...] += jnp.sum(enc, axis=0, keepdims=True)
    xo_ref[...] = x_ref[...] + quant

    @pl.when(i == nblk - 1)
    def _fin():
        loss_ref[...] = jnp.full((1, 1), CC * (ll_acc[0, 0] / N_NODES),
                                 jnp.float32)
        avg = cnt_acc[...] / N_NODES
        perp_ref[...] = jnp.full(
            (1, 1), jnp.exp(-jnp.sum(avg * jnp.log(avg + 1e-10))), jnp.float32)


def _vq_stage(conv_raw, X, gum, cb, conv_b, gg, gb, gw, gbias):
    xo, loss, perp = pl.pallas_call(
        _vq_body,
        grid=(N_NODES // R_B,),
        in_specs=[
            pl.BlockSpec((R_B, D), lambda i: (i, 0)),
            pl.BlockSpec((R_B, D), lambda i: (i, 0)),
            pl.BlockSpec((R_B, K), lambda i: (i, 0)),
            pl.BlockSpec((K, D), lambda i: (0, 0)),
            pl.BlockSpec((1, D), lambda i: (0, 0)),
            pl.BlockSpec((1, D), lambda i: (0, 0)),
            pl.BlockSpec((1, D), lambda i: (0, 0)),
            pl.BlockSpec((1, D), lambda i: (0, 0)),
            pl.BlockSpec((1, 1), lambda i: (0, 0)),
        ],
        out_specs=[
            pl.BlockSpec((R_B, D), lambda i: (i, 0)),
            pl.BlockSpec((1, 1), lambda i: (0, 0)),
            pl.BlockSpec((1, 1), lambda i: (0, 0)),
        ],
        out_shape=[
            jax.ShapeDtypeStruct((N_NODES, D), jnp.float32),
            jax.ShapeDtypeStruct((1, 1), jnp.float32),
            jax.ShapeDtypeStruct((1, 1), jnp.float32),
        ],
        scratch_shapes=[
            pltpu.SMEM((1, 1), jnp.float32),
            pltpu.VMEM((1, K), jnp.float32),
        ],
    )(conv_raw, X, gum, cb, conv_b.reshape(1, D), gg.reshape(1, D),
      gb.reshape(1, D), gw.reshape(1, D), gbias.reshape(1, 1))
    return xo, loss[0, 0], perp[0, 0]


def _conv_middle(xW, src, edge, Binv, Dinv):
    m = Binv[:, None] * jax.ops.segment_sum(xW[src], edge, num_segments=N_NODES)
    out = Dinv[:, None] * jax.ops.segment_sum(m[edge], src, num_segments=N_NODES)
    return out


def _degrees(src, edge):
    ones = jnp.ones((N_INC,), jnp.float32)
    Dd = jax.ops.segment_sum(ones, src, num_segments=N_NODES)
    Bd = jax.ops.segment_sum(ones, edge, num_segments=N_NODES)
    Dinv = jnp.where(Dd > 0, 1.0 / Dd, 0.0)
    Binv = jnp.where(Bd > 0, 1.0 / Bd, 0.0)
    return Binv, Dinv


def kernel(X, H, params, codebooks):
    src, edge = H[0], H[1]
    Binv, Dinv = _degrees(src, edge)
    base = jax.random.key(42)
    loss_latents = jnp.float32(0.0)
    perp = jnp.float32(0.0)
    for i in range(L):
        p = params[i]
        gum = jax.random.gumbel(jax.random.fold_in(base, i), (N_NODES, K),
                                dtype=jnp.float32)
        xW = _conv_in(X, p['bn_g'], p['bn_b'], p['conv_W'])
        conv_raw = _conv_middle(xW, src, edge, Binv, Dinv)
        X, loss, perp = _vq_stage(conv_raw, X, gum, codebooks[i], p['conv_b'],
                                  p['gbn_g'], p['gbn_b'], p['gate_W'][:, 0],
                                  p['gate_b'])
        loss_latents = loss_latents + loss
    return X, loss_latents, perp
